# baseline (speedup 1.0000x reference)
import dataclasses
import functools

import jax
import jax.numpy as jnp
from jax import lax
from jax.experimental import pallas as pl
from jax.experimental.pallas import tpu as pltpu
from jax.experimental.pallas import tpu_sc as plsc

BATCH = 16384
DIM = 32
LANES = 16
ROWS_PER_LINE = 4
NUM_CORES = 2
NUM_SUBCORES = 16
NUM_WORKERS = NUM_CORES * NUM_SUBCORES
BPW = BATCH // NUM_WORKERS
CHUNK = 256

TC_BLOCK = 512

TR_LANES = 2048
TR_GRID = -(-1000000 // TR_LANES)


def _transpose_body(in_ref, out_ref):
    tr = jnp.swapaxes(in_ref[...], 0, 1).reshape(TR_LANES // 4, 4, DIM)
    for g in range(ROWS_PER_LINE):
        out_ref[:, g * DIM:(g + 1) * DIM] = tr[:, g, :]


def _tc_relayout(tab_t):
    return pl.pallas_call(
        _transpose_body,
        out_shape=jax.ShapeDtypeStruct((250000, 128), jnp.float32),
        grid=(TR_GRID,),
        in_specs=[pl.BlockSpec((DIM, TR_LANES), lambda i: (0, i))],
        out_specs=pl.BlockSpec((TR_LANES // 4, 128), lambda i: (i, 0)),
    )(tab_t)


def _gather_body(users_hbm, items_hbm, ut_hbm, it_hbm, uout_hbm, iout_hbm,
                 uidx_v, iidx_v, uridx_v, iridx_v, ubuf, vbuf, sem_u, sem_i):
    wid = lax.axis_index("s") * NUM_CORES + lax.axis_index("c")
    base = wid * BPW

    pltpu.sync_copy(users_hbm.at[pl.ds(base, BPW)], uidx_v)
    pltpu.sync_copy(items_hbm.at[pl.ds(base, BPW)], iidx_v)

    @pl.loop(0, BPW, step=LANES)
    def _(i0):
        uridx_v[pl.ds(i0, LANES)] = lax.shift_right_logical(
            uidx_v[pl.ds(i0, LANES)], 2)
        iridx_v[pl.ds(i0, LANES)] = lax.shift_right_logical(
            iidx_v[pl.ds(i0, LANES)], 2)

    for c in range(BPW // CHUNK):
        off = c * CHUNK
        cu = pltpu.async_copy(
            ut_hbm.at[uridx_v.at[pl.ds(off, CHUNK)]], ubuf, sem_u)
        ci = pltpu.async_copy(
            it_hbm.at[iridx_v.at[pl.ds(off, CHUNK)]], vbuf, sem_i)
        cu.wait()
        ci.wait()
        pltpu.sync_copy(ubuf, uout_hbm.at[pl.ds(base + off, CHUNK), :])
        pltpu.sync_copy(vbuf, iout_hbm.at[pl.ds(base + off, CHUNK), :])


def _sc_gather(users, items, ut, it):
    mesh = plsc.VectorSubcoreMesh(core_axis_name="c", subcore_axis_name="s")
    cp = dataclasses.replace(
        pltpu.CompilerParams(),
        needs_layout_passes=False,
        use_tc_tiling_on_sc=True,
    )
    run = pl.kernel(
        _gather_body,
        out_type=(
            jax.ShapeDtypeStruct((BATCH, 128), jnp.float32),
            jax.ShapeDtypeStruct((BATCH, 128), jnp.float32),
        ),
        mesh=mesh,
        scratch_types=[
            pltpu.VMEM((BPW,), jnp.int32),
            pltpu.VMEM((BPW,), jnp.int32),
            pltpu.VMEM((BPW,), jnp.int32),
            pltpu.VMEM((BPW,), jnp.int32),
            pltpu.VMEM((CHUNK, 128), jnp.float32),
            pltpu.VMEM((CHUNK, 128), jnp.float32),
            pltpu.SemaphoreType.DMA,
            pltpu.SemaphoreType.DMA,
        ],
        compiler_params=cp,
    )
    return run(users, items, ut, it)


def _finish_body(u_ref, v_ref, gu_ref, gv_ref, out_ref):
    gu = gu_ref[...] % ROWS_PER_LINE
    gv = gv_ref[...] % ROWS_PER_LINE
    acc = jnp.zeros((TC_BLOCK, 1), jnp.float32)
    for g in range(ROWS_PER_LINE):
        um = (gu == g).astype(jnp.float32)
        for h in range(ROWS_PER_LINE):
            vm = (gv == h).astype(jnp.float32)
            dots = jnp.sum(u_ref[:, g * DIM:(g + 1) * DIM]
                           * v_ref[:, h * DIM:(h + 1) * DIM],
                           axis=1, keepdims=True)
            acc = acc + um * vm * dots
    out_ref[...] = 1.0 / (1.0 + jnp.exp(-acc))


def _tc_finish(urows, irows, users_col, items_col):
    grid = (BATCH // TC_BLOCK,)
    return pl.pallas_call(
        _finish_body,
        out_shape=jax.ShapeDtypeStruct((BATCH, 1), jnp.float32),
        grid=grid,
        in_specs=[
            pl.BlockSpec((TC_BLOCK, 128), lambda i: (i, 0)),
            pl.BlockSpec((TC_BLOCK, 128), lambda i: (i, 0)),
            pl.BlockSpec((TC_BLOCK, 1), lambda i: (i, 0)),
            pl.BlockSpec((TC_BLOCK, 1), lambda i: (i, 0)),
        ],
        out_specs=pl.BlockSpec((TC_BLOCK, 1), lambda i: (i, 0)),
    )(urows, irows, users_col, items_col)


@jax.jit
def kernel(users, items, user_table, item_table):
    ut = _tc_relayout(user_table.T)
    it = _tc_relayout(item_table.T)
    urows, irows = _sc_gather(users, items, ut, it)
    out = _tc_finish(urows, irows,
                     users.reshape(BATCH, 1), items.reshape(BATCH, 1))
    return out.reshape(BATCH)


# speedup vs baseline: 1.7037x; 1.7037x over previous
import dataclasses

import jax
import jax.numpy as jnp
from jax import lax
from jax.experimental import pallas as pl
from jax.experimental.pallas import tpu as pltpu
from jax.experimental.pallas import tpu_sc as plsc

BATCH = 16384
DIM = 32
LANES = 16
ROWS_PER_LINE = 4
NUM_CORES = 2
NUM_SUBCORES = 16
NUM_WORKERS = NUM_CORES * NUM_SUBCORES
BPW = BATCH // NUM_WORKERS
CHUNK = 256

TC_BLOCK = 512

TR_LANES = 8192
TR_GRID = -(-1000000 // TR_LANES)
N_LINES = (-(-1000000 // 512)) * 128


def _relayout_body(in_ref, out_ref):
    for t in range(TR_LANES // 512):
        for g in range(ROWS_PER_LINE):
            src = in_ref[:, 512 * t + 128 * g: 512 * t + 128 * g + 128]
            out_ref[128 * t:128 * (t + 1), DIM * g:DIM * (g + 1)] = (
                jnp.swapaxes(src, 0, 1))


def _tc_relayout(tab_t):
    return pl.pallas_call(
        _relayout_body,
        out_shape=jax.ShapeDtypeStruct((N_LINES, 128), jnp.float32),
        grid=(TR_GRID,),
        in_specs=[pl.BlockSpec((DIM, TR_LANES), lambda i: (0, i))],
        out_specs=pl.BlockSpec((TR_LANES // 4, 128), lambda i: (i, 0)),
    )(tab_t)


def _line_of(idx):
    return lax.bitwise_or(
        lax.shift_left(lax.shift_right_logical(idx, 9), 7),
        lax.bitwise_and(idx, 127))


def _gather_body(users_hbm, items_hbm, ut_hbm, it_hbm, uout_hbm, iout_hbm,
                 uidx_v, iidx_v, uridx_v, iridx_v, ubuf, vbuf, sem_u, sem_i):
    wid = lax.axis_index("s") * NUM_CORES + lax.axis_index("c")
    base = wid * BPW

    pltpu.sync_copy(users_hbm.at[pl.ds(base, BPW)], uidx_v)
    pltpu.sync_copy(items_hbm.at[pl.ds(base, BPW)], iidx_v)

    @pl.loop(0, BPW, step=LANES)
    def _(i0):
        uridx_v[pl.ds(i0, LANES)] = _line_of(uidx_v[pl.ds(i0, LANES)])
        iridx_v[pl.ds(i0, LANES)] = _line_of(iidx_v[pl.ds(i0, LANES)])

    for c in range(BPW // CHUNK):
        off = c * CHUNK
        cu = pltpu.async_copy(
            ut_hbm.at[uridx_v.at[pl.ds(off, CHUNK)]], ubuf, sem_u)
        ci = pltpu.async_copy(
            it_hbm.at[iridx_v.at[pl.ds(off, CHUNK)]], vbuf, sem_i)
        cu.wait()
        ci.wait()
        pltpu.sync_copy(ubuf, uout_hbm.at[pl.ds(base + off, CHUNK), :])
        pltpu.sync_copy(vbuf, iout_hbm.at[pl.ds(base + off, CHUNK), :])


def _sc_gather(users, items, ut, it):
    mesh = plsc.VectorSubcoreMesh(core_axis_name="c", subcore_axis_name="s")
    cp = dataclasses.replace(
        pltpu.CompilerParams(),
        needs_layout_passes=False,
        use_tc_tiling_on_sc=True,
    )
    run = pl.kernel(
        _gather_body,
        out_type=(
            jax.ShapeDtypeStruct((BATCH, 128), jnp.float32),
            jax.ShapeDtypeStruct((BATCH, 128), jnp.float32),
        ),
        mesh=mesh,
        scratch_types=[
            pltpu.VMEM((BPW,), jnp.int32),
            pltpu.VMEM((BPW,), jnp.int32),
            pltpu.VMEM((BPW,), jnp.int32),
            pltpu.VMEM((BPW,), jnp.int32),
            pltpu.VMEM((CHUNK, 128), jnp.float32),
            pltpu.VMEM((CHUNK, 128), jnp.float32),
            pltpu.SemaphoreType.DMA,
            pltpu.SemaphoreType.DMA,
        ],
        compiler_params=cp,
    )
    return run(users, items, ut, it)


def _finish_body(u_ref, v_ref, gu_ref, gv_ref, out_ref):
    gu = lax.shift_right_logical(gu_ref[...], 7) % ROWS_PER_LINE
    gv = lax.shift_right_logical(gv_ref[...], 7) % ROWS_PER_LINE
    acc = jnp.zeros((TC_BLOCK, 1), jnp.float32)
    for g in range(ROWS_PER_LINE):
        um = (gu == g).astype(jnp.float32)
        for h in range(ROWS_PER_LINE):
            vm = (gv == h).astype(jnp.float32)
            dots = jnp.sum(u_ref[:, g * DIM:(g + 1) * DIM]
                           * v_ref[:, h * DIM:(h + 1) * DIM],
                           axis=1, keepdims=True)
            acc = acc + um * vm * dots
    out_ref[...] = 1.0 / (1.0 + jnp.exp(-acc))


def _tc_finish(urows, irows, users_col, items_col):
    return pl.pallas_call(
        _finish_body,
        out_shape=jax.ShapeDtypeStruct((BATCH, 1), jnp.float32),
        grid=(BATCH // TC_BLOCK,),
        in_specs=[
            pl.BlockSpec((TC_BLOCK, 128), lambda i: (i, 0)),
            pl.BlockSpec((TC_BLOCK, 128), lambda i: (i, 0)),
            pl.BlockSpec((TC_BLOCK, 1), lambda i: (i, 0)),
            pl.BlockSpec((TC_BLOCK, 1), lambda i: (i, 0)),
        ],
        out_specs=pl.BlockSpec((TC_BLOCK, 1), lambda i: (i, 0)),
    )(urows, irows, users_col, items_col)


@jax.jit
def kernel(users, items, user_table, item_table):
    ut = _tc_relayout(user_table.T)
    it = _tc_relayout(item_table.T)
    urows, irows = _sc_gather(users, items, ut, it)
    out = _tc_finish(urows, irows,
                     users.reshape(BATCH, 1), items.reshape(BATCH, 1))
    return out.reshape(BATCH)


# speedup vs baseline: 1.7071x; 1.0020x over previous
import dataclasses

import jax
import jax.numpy as jnp
from jax import lax
from jax.experimental import pallas as pl
from jax.experimental.pallas import tpu as pltpu
from jax.experimental.pallas import tpu_sc as plsc

BATCH = 16384
DIM = 32
LANES = 16
ROWS_PER_LINE = 4
NUM_CORES = 2
NUM_SUBCORES = 16
NUM_WORKERS = NUM_CORES * NUM_SUBCORES
BPW = BATCH // NUM_WORKERS
CHUNK = 256

TC_BLOCK = 512

TR_LANES = 8192
TR_GRID = -(-1000000 // TR_LANES)
N_LINES = (-(-1000000 // 512)) * 128


def _relayout_body(in_ref, out_ref):
    tr = jnp.swapaxes(in_ref[...], 0, 1)
    for t in range(TR_LANES // 512):
        for g in range(ROWS_PER_LINE):
            out_ref[128 * t:128 * (t + 1), DIM * g:DIM * (g + 1)] = (
                tr[512 * t + 128 * g: 512 * t + 128 * (g + 1), :])


def _tc_relayout(tab_t):
    return pl.pallas_call(
        _relayout_body,
        out_shape=jax.ShapeDtypeStruct((N_LINES, 128), jnp.float32),
        grid=(TR_GRID,),
        in_specs=[pl.BlockSpec((DIM, TR_LANES), lambda i: (0, i))],
        out_specs=pl.BlockSpec((TR_LANES // 4, 128), lambda i: (i, 0)),
        compiler_params=pltpu.CompilerParams(
            dimension_semantics=("parallel",)),
    )(tab_t)


def _line_of(idx):
    return lax.bitwise_or(
        lax.shift_left(lax.shift_right_logical(idx, 9), 7),
        lax.bitwise_and(idx, 127))


def _gather_body(users_hbm, items_hbm, ut_hbm, it_hbm, uout_hbm, iout_hbm,
                 uidx_v, iidx_v, uridx_v, iridx_v, ubuf, vbuf, sem_u, sem_i):
    wid = lax.axis_index("s") * NUM_CORES + lax.axis_index("c")
    base = wid * BPW

    pltpu.sync_copy(users_hbm.at[pl.ds(base, BPW)], uidx_v)
    pltpu.sync_copy(items_hbm.at[pl.ds(base, BPW)], iidx_v)

    @pl.loop(0, BPW, step=LANES)
    def _(i0):
        uridx_v[pl.ds(i0, LANES)] = _line_of(uidx_v[pl.ds(i0, LANES)])
        iridx_v[pl.ds(i0, LANES)] = _line_of(iidx_v[pl.ds(i0, LANES)])

    for c in range(BPW // CHUNK):
        off = c * CHUNK
        cu = pltpu.async_copy(
            ut_hbm.at[uridx_v.at[pl.ds(off, CHUNK)]], ubuf, sem_u)
        ci = pltpu.async_copy(
            it_hbm.at[iridx_v.at[pl.ds(off, CHUNK)]], vbuf, sem_i)
        cu.wait()
        ci.wait()
        pltpu.sync_copy(ubuf, uout_hbm.at[pl.ds(base + off, CHUNK), :])
        pltpu.sync_copy(vbuf, iout_hbm.at[pl.ds(base + off, CHUNK), :])


def _sc_gather(users, items, ut, it):
    mesh = plsc.VectorSubcoreMesh(core_axis_name="c", subcore_axis_name="s")
    cp = dataclasses.replace(
        pltpu.CompilerParams(),
        needs_layout_passes=False,
        use_tc_tiling_on_sc=True,
    )
    run = pl.kernel(
        _gather_body,
        out_type=(
            jax.ShapeDtypeStruct((BATCH, 128), jnp.float32),
            jax.ShapeDtypeStruct((BATCH, 128), jnp.float32),
        ),
        mesh=mesh,
        scratch_types=[
            pltpu.VMEM((BPW,), jnp.int32),
            pltpu.VMEM((BPW,), jnp.int32),
            pltpu.VMEM((BPW,), jnp.int32),
            pltpu.VMEM((BPW,), jnp.int32),
            pltpu.VMEM((CHUNK, 128), jnp.float32),
            pltpu.VMEM((CHUNK, 128), jnp.float32),
            pltpu.SemaphoreType.DMA,
            pltpu.SemaphoreType.DMA,
        ],
        compiler_params=cp,
    )
    return run(users, items, ut, it)


def _finish_body(u_ref, v_ref, gu_ref, gv_ref, out_ref):
    gu = lax.shift_right_logical(gu_ref[...], 7) % ROWS_PER_LINE
    gv = lax.shift_right_logical(gv_ref[...], 7) % ROWS_PER_LINE
    acc = jnp.zeros((TC_BLOCK, 1), jnp.float32)
    for g in range(ROWS_PER_LINE):
        um = (gu == g).astype(jnp.float32)
        for h in range(ROWS_PER_LINE):
            vm = (gv == h).astype(jnp.float32)
            dots = jnp.sum(u_ref[:, g * DIM:(g + 1) * DIM]
                           * v_ref[:, h * DIM:(h + 1) * DIM],
                           axis=1, keepdims=True)
            acc = acc + um * vm * dots
    out_ref[...] = 1.0 / (1.0 + jnp.exp(-acc))


def _tc_finish(urows, irows, users_col, items_col):
    return pl.pallas_call(
        _finish_body,
        out_shape=jax.ShapeDtypeStruct((BATCH, 1), jnp.float32),
        grid=(BATCH // TC_BLOCK,),
        in_specs=[
            pl.BlockSpec((TC_BLOCK, 128), lambda i: (i, 0)),
            pl.BlockSpec((TC_BLOCK, 128), lambda i: (i, 0)),
            pl.BlockSpec((TC_BLOCK, 1), lambda i: (i, 0)),
            pl.BlockSpec((TC_BLOCK, 1), lambda i: (i, 0)),
        ],
        out_specs=pl.BlockSpec((TC_BLOCK, 1), lambda i: (i, 0)),
    )(urows, irows, users_col, items_col)


@jax.jit
def kernel(users, items, user_table, item_table):
    ut = _tc_relayout(user_table.T)
    it = _tc_relayout(item_table.T)
    urows, irows = _sc_gather(users, items, ut, it)
    out = _tc_finish(urows, irows,
                     users.reshape(BATCH, 1), items.reshape(BATCH, 1))
    return out.reshape(BATCH)


# speedup vs baseline: 3.0062x; 1.7610x over previous
import dataclasses

import jax
import jax.numpy as jnp
from jax import lax
from jax.experimental import pallas as pl
from jax.experimental.pallas import tpu as pltpu
from jax.experimental.pallas import tpu_sc as plsc

BATCH = 16384
DIM = 32
LANES = 16
ROWS_PER_LINE = 4
NUM_CORES = 2
NUM_SUBCORES = 16
NUM_WORKERS = NUM_CORES * NUM_SUBCORES
BPW = BATCH // NUM_WORKERS
CHUNK = 256

TC_BLOCK = 512

TR_LANES = 8192
TR_GRID = -(-1000000 // TR_LANES)
QUARTER = TR_LANES // 4
N_LINES = TR_GRID * QUARTER


def _relayout_body(in_ref, out_ref):
    st = jnp.concatenate(
        [in_ref[:, QUARTER * a: QUARTER * (a + 1)]
         for a in range(ROWS_PER_LINE)], axis=0)
    out_ref[...] = jnp.swapaxes(st, 0, 1)


def _tc_relayout(tab_t):
    return pl.pallas_call(
        _relayout_body,
        out_shape=jax.ShapeDtypeStruct((N_LINES, 128), jnp.float32),
        grid=(TR_GRID,),
        in_specs=[pl.BlockSpec((DIM, TR_LANES), lambda i: (0, i))],
        out_specs=pl.BlockSpec((TR_LANES // 4, 128), lambda i: (i, 0)),
        compiler_params=pltpu.CompilerParams(
            dimension_semantics=("parallel",)),
    )(tab_t)


def _line_of(idx):
    return lax.bitwise_or(
        lax.shift_left(lax.shift_right_logical(idx, 13), 11),
        lax.bitwise_and(idx, 2047))


def _gather_body(users_hbm, items_hbm, ut_hbm, it_hbm, uout_hbm, iout_hbm,
                 uidx_v, iidx_v, uridx_v, iridx_v, ubuf, vbuf, sem_u, sem_i):
    wid = lax.axis_index("s") * NUM_CORES + lax.axis_index("c")
    base = wid * BPW

    pltpu.sync_copy(users_hbm.at[pl.ds(base, BPW)], uidx_v)
    pltpu.sync_copy(items_hbm.at[pl.ds(base, BPW)], iidx_v)

    @pl.loop(0, BPW, step=LANES)
    def _(i0):
        uridx_v[pl.ds(i0, LANES)] = _line_of(uidx_v[pl.ds(i0, LANES)])
        iridx_v[pl.ds(i0, LANES)] = _line_of(iidx_v[pl.ds(i0, LANES)])

    for c in range(BPW // CHUNK):
        off = c * CHUNK
        cu = pltpu.async_copy(
            ut_hbm.at[uridx_v.at[pl.ds(off, CHUNK)]], ubuf, sem_u)
        ci = pltpu.async_copy(
            it_hbm.at[iridx_v.at[pl.ds(off, CHUNK)]], vbuf, sem_i)
        cu.wait()
        ci.wait()
        pltpu.sync_copy(ubuf, uout_hbm.at[pl.ds(base + off, CHUNK), :])
        pltpu.sync_copy(vbuf, iout_hbm.at[pl.ds(base + off, CHUNK), :])


def _sc_gather(users, items, ut, it):
    mesh = plsc.VectorSubcoreMesh(core_axis_name="c", subcore_axis_name="s")
    cp = dataclasses.replace(
        pltpu.CompilerParams(),
        needs_layout_passes=False,
        use_tc_tiling_on_sc=True,
    )
    run = pl.kernel(
        _gather_body,
        out_type=(
            jax.ShapeDtypeStruct((BATCH, 128), jnp.float32),
            jax.ShapeDtypeStruct((BATCH, 128), jnp.float32),
        ),
        mesh=mesh,
        scratch_types=[
            pltpu.VMEM((BPW,), jnp.int32),
            pltpu.VMEM((BPW,), jnp.int32),
            pltpu.VMEM((BPW,), jnp.int32),
            pltpu.VMEM((BPW,), jnp.int32),
            pltpu.VMEM((CHUNK, 128), jnp.float32),
            pltpu.VMEM((CHUNK, 128), jnp.float32),
            pltpu.SemaphoreType.DMA,
            pltpu.SemaphoreType.DMA,
        ],
        compiler_params=cp,
    )
    return run(users, items, ut, it)


def _finish_body(u_ref, v_ref, gu_ref, gv_ref, out_ref):
    gu = lax.shift_right_logical(gu_ref[...], 11) % ROWS_PER_LINE
    gv = lax.shift_right_logical(gv_ref[...], 11) % ROWS_PER_LINE
    usel = jnp.zeros((TC_BLOCK, DIM), jnp.float32)
    vsel = jnp.zeros((TC_BLOCK, DIM), jnp.float32)
    for g in range(ROWS_PER_LINE):
        usel = jnp.where(gu == g, u_ref[:, g * DIM:(g + 1) * DIM], usel)
        vsel = jnp.where(gv == g, v_ref[:, g * DIM:(g + 1) * DIM], vsel)
    dots = jnp.sum(usel * vsel, axis=1, keepdims=True)
    out_ref[...] = 1.0 / (1.0 + jnp.exp(-dots))


def _tc_finish(urows, irows, users_col, items_col):
    return pl.pallas_call(
        _finish_body,
        out_shape=jax.ShapeDtypeStruct((BATCH, 1), jnp.float32),
        grid=(BATCH // TC_BLOCK,),
        in_specs=[
            pl.BlockSpec((TC_BLOCK, 128), lambda i: (i, 0)),
            pl.BlockSpec((TC_BLOCK, 128), lambda i: (i, 0)),
            pl.BlockSpec((TC_BLOCK, 1), lambda i: (i, 0)),
            pl.BlockSpec((TC_BLOCK, 1), lambda i: (i, 0)),
        ],
        out_specs=pl.BlockSpec((TC_BLOCK, 1), lambda i: (i, 0)),
    )(urows, irows, users_col, items_col)


@jax.jit
def kernel(users, items, user_table, item_table):
    ut = _tc_relayout(user_table.T)
    it = _tc_relayout(item_table.T)
    urows, irows = _sc_gather(users, items, ut, it)
    out = _tc_finish(urows, irows,
                     users.reshape(BATCH, 1), items.reshape(BATCH, 1))
    return out.reshape(BATCH)


# speedup vs baseline: 3.8955x; 1.2958x over previous
import dataclasses

import jax
import jax.numpy as jnp
from jax import lax
from jax.experimental import pallas as pl
from jax.experimental.pallas import tpu as pltpu
from jax.experimental.pallas import tpu_sc as plsc

BATCH = 16384
DIM = 32
LANES = 16
ROWS_PER_LINE = 4
NUM_CORES = 2
NUM_SUBCORES = 16
NUM_WORKERS = NUM_CORES * NUM_SUBCORES
BPW = BATCH // NUM_WORKERS
CHUNK = 256

TC_BLOCK = 2048

TR_LANES = 16384
TR_GRID = -(-1000000 // TR_LANES)
QUARTER = TR_LANES // 4
N_LINES = TR_GRID * QUARTER


def _relayout_body(in_ref, out_ref):
    st = jnp.concatenate(
        [in_ref[:, QUARTER * a: QUARTER * (a + 1)]
         for a in range(ROWS_PER_LINE)], axis=0)
    out_ref[...] = jnp.swapaxes(st, 0, 1)


def _tc_relayout(tab_t):
    return pl.pallas_call(
        _relayout_body,
        out_shape=jax.ShapeDtypeStruct((N_LINES, 128), jnp.float32),
        grid=(TR_GRID,),
        in_specs=[pl.BlockSpec((DIM, TR_LANES), lambda i: (0, i))],
        out_specs=pl.BlockSpec((TR_LANES // 4, 128), lambda i: (i, 0)),
        compiler_params=pltpu.CompilerParams(
            dimension_semantics=("parallel",)),
    )(tab_t)


def _line_of(idx):
    return lax.bitwise_or(
        lax.shift_left(lax.shift_right_logical(idx, 14), 12),
        lax.bitwise_and(idx, QUARTER - 1))


def _gather_body(idx_hbm, tab_hbm, out_hbm, idx_v, ridx_v, buf0, buf1, sem0,
                 sem1):
    wid = lax.axis_index("s") * NUM_CORES + lax.axis_index("c")
    base = wid * BPW

    pltpu.sync_copy(idx_hbm.at[pl.ds(base, BPW)], idx_v)

    @pl.loop(0, BPW, step=LANES)
    def _(i0):
        ridx_v[pl.ds(i0, LANES)] = _line_of(idx_v[pl.ds(i0, LANES)])

    c0 = pltpu.async_copy(tab_hbm.at[ridx_v.at[pl.ds(0, CHUNK)]], buf0, sem0)
    c1 = pltpu.async_copy(
        tab_hbm.at[ridx_v.at[pl.ds(CHUNK, CHUNK)]], buf1, sem1)
    c0.wait()
    pltpu.sync_copy(buf0, out_hbm.at[pl.ds(base, CHUNK), :])
    c1.wait()
    pltpu.sync_copy(buf1, out_hbm.at[pl.ds(base + CHUNK, CHUNK), :])


def _sc_gather(idx, tab):
    mesh = plsc.VectorSubcoreMesh(core_axis_name="c", subcore_axis_name="s")
    cp = dataclasses.replace(
        pltpu.CompilerParams(),
        needs_layout_passes=False,
        use_tc_tiling_on_sc=True,
    )
    run = pl.kernel(
        _gather_body,
        out_type=jax.ShapeDtypeStruct((BATCH, 128), jnp.float32),
        mesh=mesh,
        scratch_types=[
            pltpu.VMEM((BPW,), jnp.int32),
            pltpu.VMEM((BPW,), jnp.int32),
            pltpu.VMEM((CHUNK, 128), jnp.float32),
            pltpu.VMEM((CHUNK, 128), jnp.float32),
            pltpu.SemaphoreType.DMA,
            pltpu.SemaphoreType.DMA,
        ],
        compiler_params=cp,
    )
    return run(idx, tab)


def _finish_body(u_ref, v_ref, gu_ref, gv_ref, out_ref):
    gu = lax.shift_right_logical(gu_ref[...], 12) % ROWS_PER_LINE
    gv = lax.shift_right_logical(gv_ref[...], 12) % ROWS_PER_LINE
    usel = jnp.zeros((TC_BLOCK, DIM), jnp.float32)
    vsel = jnp.zeros((TC_BLOCK, DIM), jnp.float32)
    for g in range(ROWS_PER_LINE):
        usel = jnp.where(gu == g, u_ref[:, g * DIM:(g + 1) * DIM], usel)
        vsel = jnp.where(gv == g, v_ref[:, g * DIM:(g + 1) * DIM], vsel)
    dots = jnp.sum(usel * vsel, axis=1, keepdims=True)
    out_ref[...] = 1.0 / (1.0 + jnp.exp(-dots))


def _tc_finish(urows, irows, users_col, items_col):
    return pl.pallas_call(
        _finish_body,
        out_shape=jax.ShapeDtypeStruct((BATCH, 1), jnp.float32),
        grid=(BATCH // TC_BLOCK,),
        in_specs=[
            pl.BlockSpec((TC_BLOCK, 128), lambda i: (i, 0)),
            pl.BlockSpec((TC_BLOCK, 128), lambda i: (i, 0)),
            pl.BlockSpec((TC_BLOCK, 1), lambda i: (i, 0)),
            pl.BlockSpec((TC_BLOCK, 1), lambda i: (i, 0)),
        ],
        out_specs=pl.BlockSpec((TC_BLOCK, 1), lambda i: (i, 0)),
    )(urows, irows, users_col, items_col)


@jax.jit
def kernel(users, items, user_table, item_table):
    ut = _tc_relayout(user_table.T)
    urows = _sc_gather(users, ut)
    it = _tc_relayout(item_table.T)
    irows = _sc_gather(items, it)
    out = _tc_finish(urows, irows,
                     users.reshape(BATCH, 1), items.reshape(BATCH, 1))
    return out.reshape(BATCH)


# speedup vs baseline: 4.0919x; 1.0504x over previous
import dataclasses

import jax
import jax.numpy as jnp
from jax import lax
from jax.experimental import pallas as pl
from jax.experimental.pallas import tpu as pltpu
from jax.experimental.pallas import tpu_sc as plsc

BATCH = 16384
DIM = 32
LANES = 16
ROWS_PER_LINE = 4
NUM_CORES = 2
NUM_SUBCORES = 16
NUM_WORKERS = NUM_CORES * NUM_SUBCORES
BPW = BATCH // NUM_WORKERS
CHUNK = 256

TC_BLOCK = 2048

TR_LANES = 16384
TR_GRID = -(-1000000 // TR_LANES)
QUARTER = TR_LANES // 4
N_LINES = TR_GRID * QUARTER


def _relayout_body(in_ref, out_ref):
    st = jnp.concatenate(
        [in_ref[:, QUARTER * a: QUARTER * (a + 1)]
         for a in range(ROWS_PER_LINE)], axis=0)
    out_ref[...] = jnp.swapaxes(st, 0, 1)


def _tc_relayout(tab_t):
    return pl.pallas_call(
        _relayout_body,
        out_shape=jax.ShapeDtypeStruct((N_LINES, 128), jnp.float32),
        grid=(TR_GRID,),
        in_specs=[pl.BlockSpec((DIM, TR_LANES), lambda i: (0, i))],
        out_specs=pl.BlockSpec((TR_LANES // 4, 128), lambda i: (i, 0)),
        compiler_params=pltpu.CompilerParams(
            dimension_semantics=("parallel",)),
    )(tab_t)


def _line_of(idx):
    return lax.bitwise_or(
        lax.shift_left(lax.shift_right_logical(idx, 14), 12),
        lax.bitwise_and(idx, QUARTER - 1))


def _gather_body(idx_hbm, tab_hbm, out_hbm, idx_v, ridx_v, buf0, buf1, sem0,
                 sem1):
    wid = lax.axis_index("s") * NUM_CORES + lax.axis_index("c")
    base = wid * BPW

    pltpu.sync_copy(idx_hbm.at[pl.ds(base, BPW)], idx_v)

    @pl.loop(0, BPW, step=LANES)
    def _(i0):
        ridx_v[pl.ds(i0, LANES)] = _line_of(idx_v[pl.ds(i0, LANES)])

    c0 = pltpu.async_copy(tab_hbm.at[ridx_v.at[pl.ds(0, CHUNK)]], buf0, sem0)
    c1 = pltpu.async_copy(
        tab_hbm.at[ridx_v.at[pl.ds(CHUNK, CHUNK)]], buf1, sem1)
    c0.wait()
    pltpu.sync_copy(buf0, out_hbm.at[pl.ds(base, CHUNK), :])
    c1.wait()
    pltpu.sync_copy(buf1, out_hbm.at[pl.ds(base + CHUNK, CHUNK), :])


def _sc_gather(idx, tab):
    mesh = plsc.VectorSubcoreMesh(core_axis_name="c", subcore_axis_name="s")
    cp = dataclasses.replace(
        pltpu.CompilerParams(),
        needs_layout_passes=False,
        use_tc_tiling_on_sc=True,
    )
    run = pl.kernel(
        _gather_body,
        out_type=jax.ShapeDtypeStruct((BATCH, 128), jnp.float32),
        mesh=mesh,
        scratch_types=[
            pltpu.VMEM((BPW,), jnp.int32),
            pltpu.VMEM((BPW,), jnp.int32),
            pltpu.VMEM((CHUNK, 128), jnp.float32),
            pltpu.VMEM((CHUNK, 128), jnp.float32),
            pltpu.SemaphoreType.DMA,
            pltpu.SemaphoreType.DMA,
        ],
        compiler_params=cp,
    )
    return run(idx, tab)


def _finish_body(u_ref, v_ref, gu_ref, gv_ref, out_ref):
    def select(rows, idx_col):
        g = lax.shift_right_logical(idx_col, 12) % ROWS_PER_LINE
        half = jnp.where(g >= 2, rows[:, 2 * DIM:4 * DIM],
                         rows[:, 0:2 * DIM])
        return jnp.where(g % 2 == 1, half[:, DIM:2 * DIM], half[:, 0:DIM])

    usel = select(u_ref[...], gu_ref[...])
    vsel = select(v_ref[...], gv_ref[...])
    dots = jnp.sum(usel * vsel, axis=1, keepdims=True)
    out_ref[...] = 1.0 / (1.0 + jnp.exp(-dots))


def _tc_finish(urows, irows, users_col, items_col):
    return pl.pallas_call(
        _finish_body,
        out_shape=jax.ShapeDtypeStruct((BATCH, 1), jnp.float32),
        grid=(BATCH // TC_BLOCK,),
        in_specs=[
            pl.BlockSpec((TC_BLOCK, 128), lambda i: (i, 0)),
            pl.BlockSpec((TC_BLOCK, 128), lambda i: (i, 0)),
            pl.BlockSpec((TC_BLOCK, 1), lambda i: (i, 0)),
            pl.BlockSpec((TC_BLOCK, 1), lambda i: (i, 0)),
        ],
        out_specs=pl.BlockSpec((TC_BLOCK, 1), lambda i: (i, 0)),
    )(urows, irows, users_col, items_col)


@jax.jit
def kernel(users, items, user_table, item_table):
    ut = _tc_relayout(user_table.T)
    urows = _sc_gather(users, ut)
    it = _tc_relayout(item_table.T)
    irows = _sc_gather(items, it)
    out = _tc_finish(urows, irows,
                     users.reshape(BATCH, 1), items.reshape(BATCH, 1))
    return out.reshape(BATCH)
